# Initial kernel scaffold; baseline (speedup 1.0000x reference)
import jax
import jax.numpy as jnp
from jax.experimental import pallas as pl


def kernel(X, edge_weight, edge_attr, W1, b1, W2, b2, W3, b3, Wo0, Wo1, Wo2, Wo3, Wt0, Wt1, Wt2, edge_index):
    raise NotImplementedError("write your pallas kernel here")


# speedup vs baseline: 1.2671x; 1.2671x over previous
import functools

import jax
import jax.numpy as jnp
import numpy as np
from jax.experimental import pallas as pl
from jax.experimental.pallas import tpu as pltpu

_H = 64
_CUT = 5.0
_TPERM = (0, 3, 6, 1, 4, 7, 2, 5, 8)
_DIAG = (0, 4, 8)


def _pick_block(n, target):
    for d in range(min(n, target), 0, -1):
        if n % d == 0:
            return d
    return n


def _silu(x):
    return x * jax.nn.sigmoid(x)


def _edge_mlp_kernel(ew_ref, ea_ref, w1t_ref, b1_ref, w2t_ref, b2_ref,
                     w3t_ref, b3_ref, a_ref, b_ref, c_ref):
    x = ea_ref[...]
    h1 = _silu(jnp.dot(x, w1t_ref[...], preferred_element_type=jnp.float32)
               + b1_ref[...])
    h2 = _silu(jnp.dot(h1, w2t_ref[...], preferred_element_type=jnp.float32)
               + b2_ref[...])
    g = _silu(jnp.dot(h2, w3t_ref[...], preferred_element_type=jnp.float32)
              + b3_ref[...])
    w = ew_ref[...]
    cut = 0.5 * (jnp.cos(w * (np.pi / _CUT)) + 1.0)
    cut = cut * (w < _CUT).astype(jnp.float32)
    f = g * cut
    f0 = f[:, :_H]
    f1 = f[:, _H:2 * _H]
    f2 = f[:, 2 * _H:]
    a_ref[...] = 0.5 * (f1 + f2)
    b_ref[...] = -0.5 * (f1 - f2)
    c_ref[...] = (f0 - f2) * (1.0 / 3.0)


def _node_pre_kernel(x_ref, wat_ref, wbt_ref, wdt_ref, xn_ref, y_ref):
    x = x_ref[...]
    tn = 0.0
    for q in range(9):
        xq = x[:, q * _H:(q + 1) * _H]
        tn = tn + xq * xq
    inv = 1.0 / (tn + 1.0)
    xn = x * jnp.concatenate([inv] * 9, axis=1)
    xn_ref[...] = xn
    wat = wat_ref[...]
    wbt = wbt_ref[...]
    dm = (xn[:, 0:_H] + xn[:, 4 * _H:5 * _H] + xn[:, 8 * _H:9 * _H]) * (1.0 / 3.0)
    diag = jnp.dot(dm, wdt_ref[...], preferred_element_type=jnp.float32)
    parts = []
    for q in range(9):
        xq = xn[:, q * _H:(q + 1) * _H]
        xqt = xn[:, _TPERM[q] * _H:(_TPERM[q] + 1) * _H]
        yq = (jnp.dot(xq, wat, preferred_element_type=jnp.float32)
              + jnp.dot(xqt, wbt, preferred_element_type=jnp.float32))
        if q in _DIAG:
            yq = yq + diag
        parts.append(yq)
    y_ref[...] = jnp.concatenate(parts, axis=1)


def _edge_scatter_kernel(src_ref, dst_ref, a_ref, b_ref, c_ref, y_ref,
                         agg_ref, *, block_e):
    @pl.when(pl.program_id(0) == 0)
    def _():
        agg_ref[...] = jnp.zeros(agg_ref.shape, agg_ref.dtype)

    def body(i, carry):
        s = src_ref[0, 0, i]
        d = dst_ref[0, 0, i]
        y = y_ref[pl.ds(s, 1), :]
        av = a_ref[pl.ds(i, 1), :]
        bv = b_ref[pl.ds(i, 1), :]
        cv = c_ref[pl.ds(i, 1), :]
        tr = (y[:, 0:_H] + y[:, 4 * _H:5 * _H] + y[:, 8 * _H:9 * _H])
        ctr = cv * tr
        parts = []
        for q in range(9):
            p = (av * y[:, q * _H:(q + 1) * _H]
                 + bv * y[:, _TPERM[q] * _H:(_TPERM[q] + 1) * _H])
            if q in _DIAG:
                p = p + ctr
            parts.append(p)
        contrib = jnp.concatenate(parts, axis=1)
        agg_ref[pl.ds(d, 1), :] = agg_ref[pl.ds(d, 1), :] + contrib
        return carry

    jax.lax.fori_loop(0, block_e, body, 0)


def _node_post_kernel(agg_ref, y_ref, xn_ref, wo3t_ref, wtat_ref, wtbt_ref,
                      wtdt_ref, out_ref):
    agg = agg_ref[...]
    y = y_ref[...]

    def sl(t, q):
        return t[:, q * _H:(q + 1) * _H]

    y2 = []
    for i in range(3):
        for j in range(3):
            acc = 0.0
            for k in range(3):
                acc = acc + sl(agg, 3 * i + k) * sl(y, 3 * k + j)
                acc = acc + sl(y, 3 * i + k) * sl(agg, 3 * k + j)
            y2.append(acc)
    n1 = 0.0
    for q in range(9):
        n1 = n1 + y2[q] * y2[q]
    inv = 1.0 / (n1 + 1.0)
    y2 = [t * inv for t in y2]
    wo3t = wo3t_ref[...]
    y2c = [jnp.dot(t, wo3t, preferred_element_type=jnp.float32) for t in y2]
    dm2 = (y2c[0] + y2c[4] + y2c[8]) * (1.0 / 3.0)
    diag = jnp.dot(dm2, wtdt_ref[...], preferred_element_type=jnp.float32)
    wtat = wtat_ref[...]
    wtbt = wtbt_ref[...]
    dx = []
    for q in range(9):
        t = (jnp.dot(y2c[q], wtat, preferred_element_type=jnp.float32)
             + jnp.dot(y2c[_TPERM[q]], wtbt, preferred_element_type=jnp.float32))
        if q in _DIAG:
            t = t + diag
        dx.append(t)
    out = []
    for i in range(3):
        for j in range(3):
            acc = dx[3 * i + j]
            for k in range(3):
                acc = acc + dx[3 * i + k] * dx[3 * k + j]
            out.append(acc)
    xn = xn_ref[...]
    res = [out[q] + sl(xn, q) for q in range(9)]
    out_ref[...] = jnp.concatenate(res, axis=1)


def kernel(X, edge_weight, edge_attr, W1, b1, W2, b2, W3, b3,
           Wo0, Wo1, Wo2, Wo3, Wt0, Wt1, Wt2, edge_index):
    N, H = X.shape[0], X.shape[1]
    E = edge_attr.shape[0]
    NR = edge_attr.shape[1]
    D = 9 * H
    f32 = jnp.float32

    w1t = W1.T
    w2t = W2.T
    w3r = W3.reshape(H, 3, 2 * H).transpose(1, 0, 2).reshape(3 * H, 2 * H)
    b3r = b3.reshape(H, 3).T.reshape(1, 3 * H)
    w3t = w3r.T
    b1r = b1.reshape(1, H)
    b2r = b2.reshape(1, 2 * H)
    wat = (0.5 * (Wo1 + Wo2)).T
    wbt = (0.5 * (Wo2 - Wo1)).T
    wdt = (Wo0 - Wo2).T
    wo3t = Wo3.T
    wtat = (0.5 * (Wt1 + Wt2)).T
    wtbt = (0.5 * (Wt2 - Wt1)).T
    wtdt = (Wt0 - Wt2).T

    x9 = X.transpose(0, 2, 3, 1).reshape(N, D)
    ew = edge_weight.reshape(E, 1)
    src = edge_index[0].astype(jnp.int32)
    dst = edge_index[1].astype(jnp.int32)

    full = lambda shape: pl.BlockSpec(shape, lambda i: (0,) * len(shape))

    be1 = _pick_block(E, 2000)
    g1 = E // be1
    a, b, c = pl.pallas_call(
        _edge_mlp_kernel,
        grid=(g1,),
        in_specs=[
            pl.BlockSpec((be1, 1), lambda i: (i, 0)),
            pl.BlockSpec((be1, NR), lambda i: (i, 0)),
            full((NR, H)), full((1, H)),
            full((H, 2 * H)), full((1, 2 * H)),
            full((2 * H, 3 * H)), full((1, 3 * H)),
        ],
        out_specs=[pl.BlockSpec((be1, H), lambda i: (i, 0))] * 3,
        out_shape=[jax.ShapeDtypeStruct((E, H), f32)] * 3,
        compiler_params=pltpu.CompilerParams(
            dimension_semantics=("parallel",)),
    )(ew, edge_attr, w1t, b1r, w2t, b2r, w3t, b3r)

    bn = _pick_block(N, 1000)
    g2 = N // bn
    xn9, y9 = pl.pallas_call(
        _node_pre_kernel,
        grid=(g2,),
        in_specs=[
            pl.BlockSpec((bn, D), lambda i: (i, 0)),
            full((H, H)), full((H, H)), full((H, H)),
        ],
        out_specs=[pl.BlockSpec((bn, D), lambda i: (i, 0))] * 2,
        out_shape=[jax.ShapeDtypeStruct((N, D), f32)] * 2,
        compiler_params=pltpu.CompilerParams(
            dimension_semantics=("parallel",)),
    )(x9, wat, wbt, wdt)

    be3 = _pick_block(E, 1000)
    g3 = E // be3
    src3 = src.reshape(g3, 1, be3)
    dst3 = dst.reshape(g3, 1, be3)
    agg = pl.pallas_call(
        functools.partial(_edge_scatter_kernel, block_e=be3),
        grid=(g3,),
        in_specs=[
            pl.BlockSpec((1, 1, be3), lambda i: (i, 0, 0),
                         memory_space=pltpu.SMEM),
            pl.BlockSpec((1, 1, be3), lambda i: (i, 0, 0),
                         memory_space=pltpu.SMEM),
            pl.BlockSpec((be3, H), lambda i: (i, 0)),
            pl.BlockSpec((be3, H), lambda i: (i, 0)),
            pl.BlockSpec((be3, H), lambda i: (i, 0)),
            full((N, D)),
        ],
        out_specs=full((N, D)),
        out_shape=jax.ShapeDtypeStruct((N, D), f32),
        compiler_params=pltpu.CompilerParams(
            dimension_semantics=("arbitrary",)),
    )(src3, dst3, a, b, c, y9)

    out9 = pl.pallas_call(
        _node_post_kernel,
        grid=(g2,),
        in_specs=[
            pl.BlockSpec((bn, D), lambda i: (i, 0)),
            pl.BlockSpec((bn, D), lambda i: (i, 0)),
            pl.BlockSpec((bn, D), lambda i: (i, 0)),
            full((H, H)), full((H, H)), full((H, H)), full((H, H)),
        ],
        out_specs=pl.BlockSpec((bn, D), lambda i: (i, 0)),
        out_shape=jax.ShapeDtypeStruct((N, D), f32),
        compiler_params=pltpu.CompilerParams(
            dimension_semantics=("parallel",)),
    )(agg, y9, xn9, wo3t, wtat, wtbt, wtdt)

    return out9.reshape(N, 3, 3, H).transpose(0, 3, 1, 2)


# speedup vs baseline: 7.0491x; 5.5630x over previous
import functools

import jax
import jax.numpy as jnp
import numpy as np
from jax.experimental import pallas as pl
from jax.experimental.pallas import tpu as pltpu

_H = 64
_CUT = 5.0
_TPERM = (0, 3, 6, 1, 4, 7, 2, 5, 8)
_DIAG = (0, 4, 8)


def _pick_block(n, target):
    for d in range(min(n, target), 0, -1):
        if n % d == 0:
            return d
    return n


def _silu(x):
    return x * jax.nn.sigmoid(x)


def _edge_mlp_kernel(ew_ref, ea_ref, w1t_ref, b1_ref, w2t_ref, b2_ref,
                     w3t_ref, b3_ref, a_ref, b_ref, c_ref):
    x = ea_ref[...]
    h1 = _silu(jnp.dot(x, w1t_ref[...], preferred_element_type=jnp.float32)
               + b1_ref[...])
    h2 = _silu(jnp.dot(h1, w2t_ref[...], preferred_element_type=jnp.float32)
               + b2_ref[...])
    g = _silu(jnp.dot(h2, w3t_ref[...], preferred_element_type=jnp.float32)
              + b3_ref[...])
    w = ew_ref[...]
    cut = 0.5 * (jnp.cos(w * (np.pi / _CUT)) + 1.0)
    cut = cut * (w < _CUT).astype(jnp.float32)
    f = g * cut
    f0 = f[:, :_H]
    f1 = f[:, _H:2 * _H]
    f2 = f[:, 2 * _H:]
    a_ref[...] = 0.5 * (f1 + f2)
    b_ref[...] = -0.5 * (f1 - f2)
    c_ref[...] = (f0 - f2) * (1.0 / 3.0)


def _node_pre_kernel(x_ref, wat_ref, wbt_ref, wdt_ref, xn_ref, y_ref):
    x = x_ref[...]
    tn = 0.0
    for q in range(9):
        xq = x[:, q * _H:(q + 1) * _H]
        tn = tn + xq * xq
    inv = 1.0 / (tn + 1.0)
    xn = x * jnp.concatenate([inv] * 9, axis=1)
    xn_ref[...] = xn
    wat = wat_ref[...]
    wbt = wbt_ref[...]
    dm = (xn[:, 0:_H] + xn[:, 4 * _H:5 * _H] + xn[:, 8 * _H:9 * _H]) * (1.0 / 3.0)
    diag = jnp.dot(dm, wdt_ref[...], preferred_element_type=jnp.float32)
    parts = []
    for q in range(9):
        xq = xn[:, q * _H:(q + 1) * _H]
        xqt = xn[:, _TPERM[q] * _H:(_TPERM[q] + 1) * _H]
        yq = (jnp.dot(xq, wat, preferred_element_type=jnp.float32)
              + jnp.dot(xqt, wbt, preferred_element_type=jnp.float32))
        if q in _DIAG:
            yq = yq + diag
        parts.append(yq)
    y_ref[...] = jnp.concatenate(parts, axis=1)


def _edge_scatter_kernel(src_ref, dst_ref, a_ref, b_ref, c_ref, y_ref,
                         agg_ref, *, block_e):
    @pl.when(pl.program_id(0) == 0)
    def _():
        agg_ref[...] = jnp.zeros(agg_ref.shape, agg_ref.dtype)

    nb = 8

    def body(i0, carry):
        base = i0 * nb
        rows = [y_ref[pl.ds(src_ref[0, 0, base + j], 1), :] for j in range(nb)]
        y = jnp.concatenate(rows, axis=0)
        av = a_ref[pl.ds(base, nb), :]
        bv = b_ref[pl.ds(base, nb), :]
        cv = c_ref[pl.ds(base, nb), :]
        tr = (y[:, 0:_H] + y[:, 4 * _H:5 * _H] + y[:, 8 * _H:9 * _H])
        ctr = cv * tr
        parts = []
        for q in range(9):
            p = (av * y[:, q * _H:(q + 1) * _H]
                 + bv * y[:, _TPERM[q] * _H:(_TPERM[q] + 1) * _H])
            if q in _DIAG:
                p = p + ctr
            parts.append(p)
        contrib = jnp.concatenate(parts, axis=1)
        for j in range(nb):
            d = dst_ref[0, 0, base + j]
            agg_ref[pl.ds(d, 1), :] = (agg_ref[pl.ds(d, 1), :]
                                       + contrib[j:j + 1, :])
        return carry

    jax.lax.fori_loop(0, block_e // nb, body, 0)


def _node_post_kernel(agg_ref, y_ref, xn_ref, wo3t_ref, wtat_ref, wtbt_ref,
                      wtdt_ref, out_ref):
    agg = agg_ref[...]
    y = y_ref[...]

    def sl(t, q):
        return t[:, q * _H:(q + 1) * _H]

    y2 = []
    for i in range(3):
        for j in range(3):
            acc = 0.0
            for k in range(3):
                acc = acc + sl(agg, 3 * i + k) * sl(y, 3 * k + j)
                acc = acc + sl(y, 3 * i + k) * sl(agg, 3 * k + j)
            y2.append(acc)
    n1 = 0.0
    for q in range(9):
        n1 = n1 + y2[q] * y2[q]
    inv = 1.0 / (n1 + 1.0)
    y2 = [t * inv for t in y2]
    wo3t = wo3t_ref[...]
    y2c = [jnp.dot(t, wo3t, preferred_element_type=jnp.float32) for t in y2]
    dm2 = (y2c[0] + y2c[4] + y2c[8]) * (1.0 / 3.0)
    diag = jnp.dot(dm2, wtdt_ref[...], preferred_element_type=jnp.float32)
    wtat = wtat_ref[...]
    wtbt = wtbt_ref[...]
    dx = []
    for q in range(9):
        t = (jnp.dot(y2c[q], wtat, preferred_element_type=jnp.float32)
             + jnp.dot(y2c[_TPERM[q]], wtbt, preferred_element_type=jnp.float32))
        if q in _DIAG:
            t = t + diag
        dx.append(t)
    out = []
    for i in range(3):
        for j in range(3):
            acc = dx[3 * i + j]
            for k in range(3):
                acc = acc + dx[3 * i + k] * dx[3 * k + j]
            out.append(acc)
    xn = xn_ref[...]
    res = [out[q] + sl(xn, q) for q in range(9)]
    out_ref[...] = jnp.concatenate(res, axis=1)


def kernel(X, edge_weight, edge_attr, W1, b1, W2, b2, W3, b3,
           Wo0, Wo1, Wo2, Wo3, Wt0, Wt1, Wt2, edge_index):
    N, H = X.shape[0], X.shape[1]
    E = edge_attr.shape[0]
    NR = edge_attr.shape[1]
    D = 9 * H
    f32 = jnp.float32

    w1t = W1.T
    w2t = W2.T
    w3r = W3.reshape(H, 3, 2 * H).transpose(1, 0, 2).reshape(3 * H, 2 * H)
    b3r = b3.reshape(H, 3).T.reshape(1, 3 * H)
    w3t = w3r.T
    b1r = b1.reshape(1, H)
    b2r = b2.reshape(1, 2 * H)
    wat = (0.5 * (Wo1 + Wo2)).T
    wbt = (0.5 * (Wo2 - Wo1)).T
    wdt = (Wo0 - Wo2).T
    wo3t = Wo3.T
    wtat = (0.5 * (Wt1 + Wt2)).T
    wtbt = (0.5 * (Wt2 - Wt1)).T
    wtdt = (Wt0 - Wt2).T

    x9 = X.transpose(0, 2, 3, 1).reshape(N, D)
    ew = edge_weight.reshape(E, 1)
    src = edge_index[0].astype(jnp.int32)
    dst = edge_index[1].astype(jnp.int32)

    full = lambda shape: pl.BlockSpec(shape, lambda i: (0,) * len(shape))

    be1 = _pick_block(E, 2000)
    g1 = E // be1
    a, b, c = pl.pallas_call(
        _edge_mlp_kernel,
        grid=(g1,),
        in_specs=[
            pl.BlockSpec((be1, 1), lambda i: (i, 0)),
            pl.BlockSpec((be1, NR), lambda i: (i, 0)),
            full((NR, H)), full((1, H)),
            full((H, 2 * H)), full((1, 2 * H)),
            full((2 * H, 3 * H)), full((1, 3 * H)),
        ],
        out_specs=[pl.BlockSpec((be1, H), lambda i: (i, 0))] * 3,
        out_shape=[jax.ShapeDtypeStruct((E, H), f32)] * 3,
        compiler_params=pltpu.CompilerParams(
            dimension_semantics=("parallel",)),
    )(ew, edge_attr, w1t, b1r, w2t, b2r, w3t, b3r)

    bn = _pick_block(N, 1000)
    g2 = N // bn
    xn9, y9 = pl.pallas_call(
        _node_pre_kernel,
        grid=(g2,),
        in_specs=[
            pl.BlockSpec((bn, D), lambda i: (i, 0)),
            full((H, H)), full((H, H)), full((H, H)),
        ],
        out_specs=[pl.BlockSpec((bn, D), lambda i: (i, 0))] * 2,
        out_shape=[jax.ShapeDtypeStruct((N, D), f32)] * 2,
        compiler_params=pltpu.CompilerParams(
            dimension_semantics=("parallel",)),
    )(x9, wat, wbt, wdt)

    be3 = _pick_block(E, 1000)
    g3 = E // be3
    src3 = src.reshape(g3, 1, be3)
    dst3 = dst.reshape(g3, 1, be3)
    agg = pl.pallas_call(
        functools.partial(_edge_scatter_kernel, block_e=be3),
        grid=(g3,),
        in_specs=[
            pl.BlockSpec((1, 1, be3), lambda i: (i, 0, 0),
                         memory_space=pltpu.SMEM),
            pl.BlockSpec((1, 1, be3), lambda i: (i, 0, 0),
                         memory_space=pltpu.SMEM),
            pl.BlockSpec((be3, H), lambda i: (i, 0)),
            pl.BlockSpec((be3, H), lambda i: (i, 0)),
            pl.BlockSpec((be3, H), lambda i: (i, 0)),
            full((N, D)),
        ],
        out_specs=full((N, D)),
        out_shape=jax.ShapeDtypeStruct((N, D), f32),
        compiler_params=pltpu.CompilerParams(
            dimension_semantics=("arbitrary",)),
    )(src3, dst3, a, b, c, y9)

    out9 = pl.pallas_call(
        _node_post_kernel,
        grid=(g2,),
        in_specs=[
            pl.BlockSpec((bn, D), lambda i: (i, 0)),
            pl.BlockSpec((bn, D), lambda i: (i, 0)),
            pl.BlockSpec((bn, D), lambda i: (i, 0)),
            full((H, H)), full((H, H)), full((H, H)), full((H, H)),
        ],
        out_specs=pl.BlockSpec((bn, D), lambda i: (i, 0)),
        out_shape=jax.ShapeDtypeStruct((N, D), f32),
        compiler_params=pltpu.CompilerParams(
            dimension_semantics=("parallel",)),
    )(agg, y9, xn9, wo3t, wtat, wtbt, wtdt)

    return out9.reshape(N, 3, 3, H).transpose(0, 3, 1, 2)
